# baseline (speedup 1.0000x reference)
import functools

import jax
import jax.numpy as jnp
from jax import lax
from jax.experimental import pallas as pl
from jax.experimental.pallas import tpu as pltpu
from jax.experimental.pallas import tpu_sc as plsc

NUM_ASSETS = 100
SUB_SIZE = 1000
VOCAB = NUM_ASSETS * SUB_SIZE
EMBED_DIM = 32
BATCH = 16384

_LANES = 16
_ICHUNK = 2048
_NROUNDS = BATCH // _ICHUNK
_OCHUNK = 2048


def _body(asset_hbm, shape_hbm, tablet_hbm, out_hbm,
          row_v, idx_v, a0_v, s0_v, a1_v, s1_v, o0_v, o1_v,
          row_sem, isem0, isem1, osem0, osem1):
    c = lax.axis_index("s") * 2 + lax.axis_index("c")

    row_copy = pltpu.async_copy(tablet_hbm.at[c], row_v, row_sem)

    abufs = ((a0_v, s0_v), (a1_v, s1_v))
    isems = (isem0, isem1)

    def start_round(r):
        a_v, s_v = abufs[r % 2]
        sem = isems[r % 2]
        b0 = r * _ICHUNK
        return (pltpu.async_copy(asset_hbm.at[pl.ds(b0, _ICHUNK)], a_v, sem),
                pltpu.async_copy(shape_hbm.at[pl.ds(b0, _ICHUNK)], s_v, sem))

    in_pending = start_round(0)
    for r in range(_NROUNDS):
        a_v, s_v = abufs[r % 2]
        for cp in in_pending:
            cp.wait()
        in_pending = start_round(r + 1) if r + 1 < _NROUNDS else ()
        b0 = r * _ICHUNK

        @plsc.parallel_loop(0, _ICHUNK // _LANES, unroll=8)
        def _compute(i, _b0=b0, _a=a_v, _s=s_v):
            off = i * _LANES
            idx_v[pl.ds(_b0 + off, _LANES)] = (
                _a[pl.ds(off, _LANES)] * SUB_SIZE + _s[pl.ds(off, _LANES)])

    row_copy.wait()

    obufs = (o0_v, o1_v)
    osems = (osem0, osem1)
    out_pending = [None, None]
    for ch in range(BATCH // _OCHUNK):
        slot = ch % 2
        buf = obufs[slot]
        if out_pending[slot] is not None:
            out_pending[slot].wait()
        base = ch * _OCHUNK

        @plsc.parallel_loop(0, _OCHUNK // _LANES, unroll=8)
        def _gather(i, _base=base, _buf=buf):
            off = i * _LANES
            idx = idx_v[pl.ds(_base + off, _LANES)]
            _buf[pl.ds(off, _LANES)] = plsc.load_gather(row_v, [idx])

        out_pending[slot] = pltpu.async_copy(
            buf, out_hbm.at[c, pl.ds(base, _OCHUNK)], osems[slot])

    out_pending[0].wait()
    out_pending[1].wait()


def kernel(asset_index, shape_index, table):
    tablet = table.T
    mesh = plsc.VectorSubcoreMesh(core_axis_name="c", subcore_axis_name="s")
    run = functools.partial(
        pl.kernel,
        mesh=mesh,
        out_type=jax.ShapeDtypeStruct((EMBED_DIM, BATCH), jnp.float32),
        scratch_types=[
            pltpu.VMEM((VOCAB,), jnp.float32),
            pltpu.VMEM((BATCH,), jnp.int32),
            pltpu.VMEM((_ICHUNK,), jnp.int32),
            pltpu.VMEM((_ICHUNK,), jnp.int32),
            pltpu.VMEM((_ICHUNK,), jnp.int32),
            pltpu.VMEM((_ICHUNK,), jnp.int32),
            pltpu.VMEM((_OCHUNK,), jnp.float32),
            pltpu.VMEM((_OCHUNK,), jnp.float32),
            pltpu.SemaphoreType.DMA,
            pltpu.SemaphoreType.DMA,
            pltpu.SemaphoreType.DMA,
            pltpu.SemaphoreType.DMA,
            pltpu.SemaphoreType.DMA,
        ],
        compiler_params=pltpu.CompilerParams(needs_layout_passes=False),
    )(_body)
    return run(asset_index, shape_index, tablet).T


# speedup vs baseline: 1.0432x; 1.0432x over previous
import functools

import jax
import jax.numpy as jnp
from jax import lax
from jax.experimental import pallas as pl
from jax.experimental.pallas import tpu as pltpu
from jax.experimental.pallas import tpu_sc as plsc

NUM_ASSETS = 100
SUB_SIZE = 1000
VOCAB = NUM_ASSETS * SUB_SIZE
EMBED_DIM = 32
BATCH = 16384

_LANES = 16
_ICHUNK = 4096
_NROUNDS = BATCH // _ICHUNK
_OCHUNK = 4096


def _body(asset_hbm, shape_hbm, tablet_hbm, out_hbm,
          row_v, a0_v, s0_v, a1_v, s1_v, o0_v, o1_v,
          row_sem, isem0, isem1, osem0, osem1):
    c = lax.axis_index("s") * 2 + lax.axis_index("c")

    abufs = ((a0_v, s0_v), (a1_v, s1_v))
    isems = (isem0, isem1)

    def start_round(r):
        a_v, s_v = abufs[r % 2]
        sem = isems[r % 2]
        b0 = r * _ICHUNK
        return (pltpu.async_copy(asset_hbm.at[pl.ds(b0, _ICHUNK)], a_v, sem),
                pltpu.async_copy(shape_hbm.at[pl.ds(b0, _ICHUNK)], s_v, sem))

    in_pending = start_round(0)
    row_copy = pltpu.async_copy(tablet_hbm.at[c], row_v, row_sem)
    row_copy.wait()

    obufs = (o0_v, o1_v)
    osems = (osem0, osem1)
    out_pending = [None, None]

    for r in range(_NROUNDS):
        a_v, s_v = abufs[r % 2]
        for cp in in_pending:
            cp.wait()
        in_pending = start_round(r + 1) if r + 1 < _NROUNDS else ()

        for ch in range(_ICHUNK // _OCHUNK):
            slot = (r * (_ICHUNK // _OCHUNK) + ch) % 2
            buf = obufs[slot]
            if out_pending[slot] is not None:
                out_pending[slot].wait()
            cbase = ch * _OCHUNK

            @plsc.parallel_loop(0, _OCHUNK // _LANES, unroll=8)
            def _gather(i, _cbase=cbase, _buf=buf, _a=a_v, _s=s_v):
                off = _cbase + i * _LANES
                idx = (_a[pl.ds(off, _LANES)] * SUB_SIZE
                       + _s[pl.ds(off, _LANES)])
                _buf[pl.ds(i * _LANES, _LANES)] = plsc.load_gather(row_v, [idx])

            out_pending[slot] = pltpu.async_copy(
                buf, out_hbm.at[c, pl.ds(r * _ICHUNK + cbase, _OCHUNK)],
                osems[slot])

    out_pending[0].wait()
    out_pending[1].wait()


def kernel(asset_index, shape_index, table):
    tablet = table.T
    mesh = plsc.VectorSubcoreMesh(core_axis_name="c", subcore_axis_name="s")
    run = functools.partial(
        pl.kernel,
        mesh=mesh,
        out_type=jax.ShapeDtypeStruct((EMBED_DIM, BATCH), jnp.float32),
        scratch_types=[
            pltpu.VMEM((VOCAB,), jnp.float32),
            pltpu.VMEM((_ICHUNK,), jnp.int32),
            pltpu.VMEM((_ICHUNK,), jnp.int32),
            pltpu.VMEM((_ICHUNK,), jnp.int32),
            pltpu.VMEM((_ICHUNK,), jnp.int32),
            pltpu.VMEM((_OCHUNK,), jnp.float32),
            pltpu.VMEM((_OCHUNK,), jnp.float32),
            pltpu.SemaphoreType.DMA,
            pltpu.SemaphoreType.DMA,
            pltpu.SemaphoreType.DMA,
            pltpu.SemaphoreType.DMA,
            pltpu.SemaphoreType.DMA,
        ],
        compiler_params=pltpu.CompilerParams(needs_layout_passes=False),
    )(_body)
    return run(asset_index, shape_index, tablet).T
